# baseline (speedup 1.0000x reference)
import functools

import jax
import jax.numpy as jnp
from jax import lax
from jax.experimental import pallas as pl
from jax.experimental.pallas import tpu as pltpu
from jax.experimental.pallas import tpu_sc as plsc

N_NODES = 10000
N_EDGES = 320000
D = 128

NC = 2
NS = 16
DH = D // NC
EPW = N_EDGES // NS
CH = 80
NCHUNK = EPW // CH
NPAIR = NCHUNK // 2
ECH = N_EDGES // CH
RPT = 624
TAIL = N_NODES - NS * RPT
ZR = 208
DEG_W = 16

f32 = jnp.float32


def _zero_vmem_rows(ref, nrows, ncols):
  npv = ncols // 16

  def body(k, _):
    i = k // npv
    j = k % npv
    ref[i, pl.ds(j * 16, 16)] = jnp.zeros((16,), f32)
    return 0

  lax.fori_loop(0, nrows * npv, body, 0)


def _make_sc_agg(with_deg):
  out_type = [jax.ShapeDtypeStruct((NC, N_NODES, DH), f32)]
  scratch = [
      pltpu.VMEM((NCHUNK, CH), jnp.int32),
      pltpu.VMEM((NCHUNK, CH), jnp.int32),
      pltpu.VMEM((2, CH, DH), f32),
      pltpu.VMEM((ZR, DH), f32),
      pltpu.VMEM_SHARED((N_NODES, DH), f32),
      pltpu.SemaphoreType.DMA,
  ]
  if with_deg:
    out_type.append(jax.ShapeDtypeStruct((NC, N_NODES, DEG_W), f32))
    scratch += [
        pltpu.VMEM((CH, DEG_W), f32),
        pltpu.VMEM((ZR, DEG_W), f32),
        pltpu.VMEM_SHARED((N_NODES, DEG_W), f32),
    ]

  mesh = plsc.VectorSubcoreMesh(core_axis_name="c", subcore_axis_name="s")

  def body(*refs):
    if with_deg:
      (y, srce, dste, outp, outd,
       src_v, dst_v, rows_v, zb, acc, sem, ones_v, zbd, accd) = refs
    else:
      (y, srce, dste, outp,
       src_v, dst_v, rows_v, zb, acc, sem) = refs

    c = lax.axis_index("c")
    s = lax.axis_index("s")

    pltpu.sync_copy(srce.at[pl.ds(s * NCHUNK, NCHUNK)], src_v)
    pltpu.sync_copy(dste.at[pl.ds(s * NCHUNK, NCHUNK)], dst_v)
    npv = CH // 16

    def remap(t, _):
      i = t // npv
      k = t % npv
      v = src_v[i, pl.ds(k * 16, 16)]
      src_v[i, pl.ds(k * 16, 16)] = v * 2 + c
      return 0

    lax.fori_loop(0, NCHUNK * npv, remap, 0)

    _zero_vmem_rows(zb, ZR, DH)
    for k in range(RPT // ZR):
      pltpu.sync_copy(zb, acc.at[pl.ds(s * RPT + k * ZR, ZR)])

    @pl.when(s == 0)
    def _():
      pltpu.sync_copy(zb.at[pl.ds(0, TAIL)], acc.at[pl.ds(NS * RPT, TAIL)])

    if with_deg:
      _zero_vmem_rows(zbd, ZR, DEG_W)
      for k in range(RPT // ZR):
        pltpu.sync_copy(zbd, accd.at[pl.ds(s * RPT + k * ZR, ZR)])

      @pl.when(s == 0)
      def _():
        pltpu.sync_copy(zbd.at[pl.ds(0, TAIL)], accd.at[pl.ds(NS * RPT, TAIL)])

      def fill_ones(i, _):
        ones_v[i] = jnp.ones((DEG_W,), f32)
        return 0

      lax.fori_loop(0, CH, fill_ones, 0)

    plsc.subcore_barrier()

    def gather(j, b):
      return pltpu.make_async_copy(y.at[src_v.at[j]], rows_v.at[b], sem)

    def scatter(j, b):
      pltpu.sync_copy(rows_v.at[b], acc.at[dst_v.at[j]], add=True)
      if with_deg:
        @pl.when(c == b)
        def _():
          pltpu.sync_copy(ones_v, accd.at[dst_v.at[j]], add=True)

    gather(0, 0).start()

    def pair(jj, _):
      j0 = 2 * jj
      gather(j0, 0).wait()
      gather(j0 + 1, 1).start()
      scatter(j0, 0)
      gather(j0 + 2, 0).start()
      gather(j0 + 1, 1).wait()
      scatter(j0 + 1, 1)
      return 0

    lax.fori_loop(0, NPAIR - 1, pair, 0)

    gather(NCHUNK - 2, 0).wait()
    gather(NCHUNK - 1, 1).start()
    scatter(NCHUNK - 2, 0)
    gather(NCHUNK - 1, 1).wait()
    scatter(NCHUNK - 1, 1)

    plsc.subcore_barrier()

    rs = s * RPT
    pltpu.sync_copy(acc.at[pl.ds(rs, RPT)], outp.at[c, pl.ds(rs, RPT)])

    @pl.when(s == 0)
    def _():
      pltpu.sync_copy(acc.at[pl.ds(NS * RPT, TAIL)],
                      outp.at[c, pl.ds(NS * RPT, TAIL)])

    if with_deg:
      pltpu.sync_copy(accd.at[pl.ds(rs, RPT)], outd.at[c, pl.ds(rs, RPT)])

      @pl.when(s == 0)
      def _():
        pltpu.sync_copy(accd.at[pl.ds(NS * RPT, TAIL)],
                        outd.at[c, pl.ds(NS * RPT, TAIL)])

  return pl.kernel(body, out_type=out_type, mesh=mesh, scratch_types=scratch,
                   compiler_params=pltpu.CompilerParams(
                       use_tc_tiling_on_sc=False),
                   name="sc_agg_deg" if with_deg else "sc_agg")


_sc_agg_deg = _make_sc_agg(True)
_sc_agg = _make_sc_agg(False)


BR = 1000


def _tc_first(x, ws, wn, b, s_out, y_out):
  xv = x[...]
  s_out[...] = jnp.dot(xv, ws[...], preferred_element_type=f32) + b[...]
  y_out[...] = jnp.dot(xv, wn[...], preferred_element_type=f32)


def _mean_from_parts(p, dr):
  deg = dr[0, :, 0:1] + dr[1, :, 0:1]
  inv = 1.0 / jnp.maximum(deg, 1.0)
  agg = jnp.concatenate([p[0], p[1]], axis=1)
  return agg * inv


def _tc_mid(s_in, p, dr, ws, wn, b, s_out, y_out):
  h = jnp.maximum(s_in[...] + _mean_from_parts(p, dr), 0.0)
  s_out[...] = jnp.dot(h, ws[...], preferred_element_type=f32) + b[...]
  y_out[...] = jnp.dot(h, wn[...], preferred_element_type=f32)


def _tc_last(s_in, p, dr, out):
  out[...] = s_in[...] + _mean_from_parts(p, dr)


_row_spec = pl.BlockSpec((BR, D), lambda i: (i, 0))
_w_spec = pl.BlockSpec((D, D), lambda i: (0, 0))
_b_spec = pl.BlockSpec((1, D), lambda i: (0, 0))
_p_spec = pl.BlockSpec((NC, BR, DH), lambda i: (0, i, 0))
_dr_spec = pl.BlockSpec((NC, BR, DEG_W), lambda i: (0, i, 0))
_grid = (N_NODES // BR,)
_nd_shape = jax.ShapeDtypeStruct((N_NODES, D), f32)

_tc_first_call = pl.pallas_call(
    _tc_first, grid=_grid,
    in_specs=[_row_spec, _w_spec, _w_spec, _b_spec],
    out_specs=[_row_spec, _row_spec],
    out_shape=[_nd_shape, _nd_shape])

_tc_mid_call = pl.pallas_call(
    _tc_mid, grid=_grid,
    in_specs=[_row_spec, _p_spec, _dr_spec, _w_spec, _w_spec, _b_spec],
    out_specs=[_row_spec, _row_spec],
    out_shape=[_nd_shape, _nd_shape])

_tc_last_call = pl.pallas_call(
    _tc_last, grid=_grid,
    in_specs=[_row_spec, _p_spec, _dr_spec],
    out_specs=_row_spec,
    out_shape=_nd_shape)


@jax.jit
def kernel(in_feat, edge_index, W_self1, W_neigh1, b1, W_self2, W_neigh2, b2,
           W_self3, W_neigh3, b3, W_self4, W_neigh4, b4):
  src = edge_index[0].reshape(ECH, CH)
  dst = edge_index[1].reshape(ECH, CH)
  s1, y1 = _tc_first_call(in_feat, W_self1, W_neigh1, b1.reshape(1, D))
  p, dr = _sc_agg_deg(y1.reshape(2 * N_NODES, DH), src, dst)
  s2, y2 = _tc_mid_call(s1, p, dr, W_self2, W_neigh2, b2.reshape(1, D))
  (p,) = _sc_agg(y2.reshape(2 * N_NODES, DH), src, dst)
  s3, y3 = _tc_mid_call(s2, p, dr, W_self3, W_neigh3, b3.reshape(1, D))
  (p,) = _sc_agg(y3.reshape(2 * N_NODES, DH), src, dst)
  s4, y4 = _tc_mid_call(s3, p, dr, W_self4, W_neigh4, b4.reshape(1, D))
  (p,) = _sc_agg(y4.reshape(2 * N_NODES, DH), src, dst)
  return _tc_last_call(s4, p, dr)


# speedup vs baseline: 1.0934x; 1.0934x over previous
import functools

import jax
import jax.numpy as jnp
from jax import lax
from jax.experimental import pallas as pl
from jax.experimental.pallas import tpu as pltpu
from jax.experimental.pallas import tpu_sc as plsc

N_NODES = 10000
N_EDGES = 320000
D = 128

NC = 2
NS = 16
DH = D // NC
EPW = N_EDGES // NS
CH = 80
NCHUNK = EPW // CH
NPAIR = NCHUNK // 2
ECH = N_EDGES // CH
RPT = 624
TAIL = N_NODES - NS * RPT
ZR = 208
DEG_W = 16

f32 = jnp.float32


def _zero_vmem_rows(ref, nrows, ncols):
  npv = ncols // 16

  def body(k, _):
    i = k // npv
    j = k % npv
    ref[i, pl.ds(j * 16, 16)] = jnp.zeros((16,), f32)
    return 0

  lax.fori_loop(0, nrows * npv, body, 0)


def _make_sc_agg(with_deg):
  out_type = [jax.ShapeDtypeStruct((NC, N_NODES, DH), f32)]
  scratch = [
      pltpu.VMEM((NCHUNK, CH), jnp.int32),
      pltpu.VMEM((NCHUNK, CH), jnp.int32),
      pltpu.VMEM((4, CH, DH), f32),
      pltpu.VMEM((ZR, DH), f32),
      pltpu.VMEM_SHARED((N_NODES, DH), f32),
      pltpu.SemaphoreType.DMA,
      pltpu.SemaphoreType.DMA,
  ]
  if with_deg:
    out_type.append(jax.ShapeDtypeStruct((NC, N_NODES, DEG_W), f32))
    scratch += [
        pltpu.VMEM((CH, DEG_W), f32),
        pltpu.VMEM((ZR, DEG_W), f32),
        pltpu.VMEM_SHARED((N_NODES, DEG_W), f32),
    ]

  mesh = plsc.VectorSubcoreMesh(core_axis_name="c", subcore_axis_name="s")

  def body(*refs):
    if with_deg:
      (y, srce, dste, outp, outd,
       src_v, dst_v, rows_v, zb, acc, gsem, ssem, ones_v, zbd, accd) = refs
    else:
      (y, srce, dste, outp,
       src_v, dst_v, rows_v, zb, acc, gsem, ssem) = refs

    c = lax.axis_index("c")
    s = lax.axis_index("s")

    pltpu.sync_copy(srce.at[pl.ds(s * NCHUNK, NCHUNK)], src_v)
    pltpu.sync_copy(dste.at[pl.ds(s * NCHUNK, NCHUNK)], dst_v)
    npv = CH // 16

    def remap(t, _):
      i = t // npv
      k = t % npv
      v = src_v[i, pl.ds(k * 16, 16)]
      src_v[i, pl.ds(k * 16, 16)] = v * 2 + c
      return 0

    lax.fori_loop(0, NCHUNK * npv, remap, 0)

    _zero_vmem_rows(zb, ZR, DH)
    for k in range(RPT // ZR):
      pltpu.sync_copy(zb, acc.at[pl.ds(s * RPT + k * ZR, ZR)])

    @pl.when(s == 0)
    def _():
      pltpu.sync_copy(zb.at[pl.ds(0, TAIL)], acc.at[pl.ds(NS * RPT, TAIL)])

    if with_deg:
      _zero_vmem_rows(zbd, ZR, DEG_W)
      for k in range(RPT // ZR):
        pltpu.sync_copy(zbd, accd.at[pl.ds(s * RPT + k * ZR, ZR)])

      @pl.when(s == 0)
      def _():
        pltpu.sync_copy(zbd.at[pl.ds(0, TAIL)], accd.at[pl.ds(NS * RPT, TAIL)])

      def fill_ones(i, _):
        ones_v[i] = jnp.ones((DEG_W,), f32)
        return 0

      lax.fori_loop(0, CH, fill_ones, 0)

    plsc.subcore_barrier()

    def gather(j, b):
      return pltpu.make_async_copy(y.at[src_v.at[j]], rows_v.at[b], gsem)

    def scat_start(j, b):
      pltpu.async_copy(rows_v.at[b], acc.at[dst_v.at[j]], ssem,
                       add=True).start()
      if with_deg:
        @pl.when(c == (b % 2))
        def _():
          pltpu.sync_copy(ones_v, accd.at[dst_v.at[j]], add=True)

    def scat_wait(j, b):
      pltpu.make_async_copy(rows_v.at[b], acc.at[dst_v.at[j]], ssem).wait()

    for b in range(4):
      gather(b, b).start()

    def ring(jj, _):
      j0 = 4 * jj
      for b in range(4):
        gather(j0 + b, b).wait()
        scat_start(j0 + b, b)
      for b in range(4):
        scat_wait(j0 + b, b)
        gather(j0 + 4 + b, b).start()
      return 0

    NRING = (NCHUNK - 6) // 4
    lax.fori_loop(0, NRING, ring, 0)

    t0 = NRING * 4
    for b in range(4):
      gather(t0 + b, b).wait()
      scat_start(t0 + b, b)
    scat_wait(t0, 0)
    gather(t0 + 4, 0).start()
    scat_wait(t0 + 1, 1)
    gather(t0 + 5, 1).start()
    gather(t0 + 4, 0).wait()
    scat_start(t0 + 4, 0)
    gather(t0 + 5, 1).wait()
    scat_start(t0 + 5, 1)
    scat_wait(t0 + 2, 2)
    scat_wait(t0 + 3, 3)
    scat_wait(t0 + 4, 0)
    scat_wait(t0 + 5, 1)

    plsc.subcore_barrier()

    rs = s * RPT
    pltpu.sync_copy(acc.at[pl.ds(rs, RPT)], outp.at[c, pl.ds(rs, RPT)])

    @pl.when(s == 0)
    def _():
      pltpu.sync_copy(acc.at[pl.ds(NS * RPT, TAIL)],
                      outp.at[c, pl.ds(NS * RPT, TAIL)])

    if with_deg:
      pltpu.sync_copy(accd.at[pl.ds(rs, RPT)], outd.at[c, pl.ds(rs, RPT)])

      @pl.when(s == 0)
      def _():
        pltpu.sync_copy(accd.at[pl.ds(NS * RPT, TAIL)],
                        outd.at[c, pl.ds(NS * RPT, TAIL)])

  return pl.kernel(body, out_type=out_type, mesh=mesh, scratch_types=scratch,
                   compiler_params=pltpu.CompilerParams(
                       use_tc_tiling_on_sc=False),
                   name="sc_agg_deg" if with_deg else "sc_agg")


_sc_agg_deg = _make_sc_agg(True)
_sc_agg = _make_sc_agg(False)


BR = 1000


def _tc_first(x, ws, wn, b, s_out, y_out):
  xv = x[...]
  s_out[...] = jnp.dot(xv, ws[...], preferred_element_type=f32) + b[...]
  y_out[...] = jnp.dot(xv, wn[...], preferred_element_type=f32)


def _mean_from_parts(p, dr):
  deg = dr[0, :, 0:1] + dr[1, :, 0:1]
  inv = 1.0 / jnp.maximum(deg, 1.0)
  agg = jnp.concatenate([p[0], p[1]], axis=1)
  return agg * inv


def _tc_mid(s_in, p, dr, ws, wn, b, s_out, y_out):
  h = jnp.maximum(s_in[...] + _mean_from_parts(p, dr), 0.0)
  s_out[...] = jnp.dot(h, ws[...], preferred_element_type=f32) + b[...]
  y_out[...] = jnp.dot(h, wn[...], preferred_element_type=f32)


def _tc_last(s_in, p, dr, out):
  out[...] = s_in[...] + _mean_from_parts(p, dr)


_row_spec = pl.BlockSpec((BR, D), lambda i: (i, 0))
_w_spec = pl.BlockSpec((D, D), lambda i: (0, 0))
_b_spec = pl.BlockSpec((1, D), lambda i: (0, 0))
_p_spec = pl.BlockSpec((NC, BR, DH), lambda i: (0, i, 0))
_dr_spec = pl.BlockSpec((NC, BR, DEG_W), lambda i: (0, i, 0))
_grid = (N_NODES // BR,)
_nd_shape = jax.ShapeDtypeStruct((N_NODES, D), f32)

_tc_first_call = pl.pallas_call(
    _tc_first, grid=_grid,
    in_specs=[_row_spec, _w_spec, _w_spec, _b_spec],
    out_specs=[_row_spec, _row_spec],
    out_shape=[_nd_shape, _nd_shape])

_tc_mid_call = pl.pallas_call(
    _tc_mid, grid=_grid,
    in_specs=[_row_spec, _p_spec, _dr_spec, _w_spec, _w_spec, _b_spec],
    out_specs=[_row_spec, _row_spec],
    out_shape=[_nd_shape, _nd_shape])

_tc_last_call = pl.pallas_call(
    _tc_last, grid=_grid,
    in_specs=[_row_spec, _p_spec, _dr_spec],
    out_specs=_row_spec,
    out_shape=_nd_shape)


@jax.jit
def kernel(in_feat, edge_index, W_self1, W_neigh1, b1, W_self2, W_neigh2, b2,
           W_self3, W_neigh3, b3, W_self4, W_neigh4, b4):
  src = edge_index[0].reshape(ECH, CH)
  dst = edge_index[1].reshape(ECH, CH)
  s1, y1 = _tc_first_call(in_feat, W_self1, W_neigh1, b1.reshape(1, D))
  p, dr = _sc_agg_deg(y1.reshape(2 * N_NODES, DH), src, dst)
  s2, y2 = _tc_mid_call(s1, p, dr, W_self2, W_neigh2, b2.reshape(1, D))
  (p,) = _sc_agg(y2.reshape(2 * N_NODES, DH), src, dst)
  s3, y3 = _tc_mid_call(s2, p, dr, W_self3, W_neigh3, b3.reshape(1, D))
  (p,) = _sc_agg(y3.reshape(2 * N_NODES, DH), src, dst)
  s4, y4 = _tc_mid_call(s3, p, dr, W_self4, W_neigh4, b4.reshape(1, D))
  (p,) = _sc_agg(y4.reshape(2 * N_NODES, DH), src, dst)
  return _tc_last_call(s4, p, dr)


# speedup vs baseline: 1.2673x; 1.1591x over previous
import functools

import jax
import jax.numpy as jnp
from jax import lax
from jax.experimental import pallas as pl
from jax.experimental.pallas import tpu as pltpu
from jax.experimental.pallas import tpu_sc as plsc

N_NODES = 10000
N_EDGES = 320000
D = 128

NC = 2
NS = 16
DH = D // NC
EPW = N_EDGES // NS
CH = 80
NCHUNK = EPW // CH
NPAIR = NCHUNK // 2
ECH = N_EDGES // CH
RPT = 624
TAIL = N_NODES - NS * RPT
ZR = 208
DEG_W = 16

f32 = jnp.float32


def _zero_vmem_rows(ref, nrows, ncols):
  npv = ncols // 16

  def body(k, _):
    i = k // npv
    j = k % npv
    ref[i, pl.ds(j * 16, 16)] = jnp.zeros((16,), f32)
    return 0

  lax.fori_loop(0, nrows * npv, body, 0)


def _make_sc_agg(with_deg):
  out_type = [jax.ShapeDtypeStruct((NC, N_NODES, DH), f32)]
  scratch = [
      pltpu.VMEM((NCHUNK, CH), jnp.int32),
      pltpu.VMEM((NCHUNK, CH), jnp.int32),
      pltpu.VMEM((4, CH, DH), f32),
      pltpu.VMEM((ZR, DH), f32),
      pltpu.VMEM_SHARED((N_NODES, DH), f32),
      pltpu.SemaphoreType.DMA,
      pltpu.SemaphoreType.DMA,
  ]
  if with_deg:
    out_type.append(jax.ShapeDtypeStruct((NC, N_NODES, DEG_W), f32))
    scratch += [
        pltpu.VMEM((CH, DEG_W), f32),
        pltpu.VMEM((ZR, DEG_W), f32),
        pltpu.VMEM_SHARED((N_NODES, DEG_W), f32),
    ]

  mesh = plsc.VectorSubcoreMesh(core_axis_name="c", subcore_axis_name="s")

  def body(*refs):
    if with_deg:
      (y, srce, dste, outp, outd,
       src_v, dst_v, rows_v, zb, acc, gsem, ssem, ones_v, zbd, accd) = refs
    else:
      (y, srce, dste, outp,
       src_v, dst_v, rows_v, zb, acc, gsem, ssem) = refs

    c = lax.axis_index("c")
    s = lax.axis_index("s")

    pltpu.sync_copy(srce.at[pl.ds(s * NCHUNK, NCHUNK)], src_v)
    pltpu.sync_copy(dste.at[pl.ds(s * NCHUNK, NCHUNK)], dst_v)
    npv = CH // 16

    def remap(t, _):
      i = t // npv
      k = t % npv
      v = src_v[i, pl.ds(k * 16, 16)]
      src_v[i, pl.ds(k * 16, 16)] = v * 2 + c
      return 0

    lax.fori_loop(0, NCHUNK * npv, remap, 0)

    _zero_vmem_rows(zb, ZR, DH)
    for k in range(RPT // ZR):
      pltpu.sync_copy(zb, acc.at[pl.ds(s * RPT + k * ZR, ZR)])

    @pl.when(s == 0)
    def _():
      pltpu.sync_copy(zb.at[pl.ds(0, TAIL)], acc.at[pl.ds(NS * RPT, TAIL)])

    if with_deg:
      _zero_vmem_rows(zbd, ZR, DEG_W)
      for k in range(RPT // ZR):
        pltpu.sync_copy(zbd, accd.at[pl.ds(s * RPT + k * ZR, ZR)])

      @pl.when(s == 0)
      def _():
        pltpu.sync_copy(zbd.at[pl.ds(0, TAIL)], accd.at[pl.ds(NS * RPT, TAIL)])

      def fill_ones(i, _):
        ones_v[i] = jnp.ones((DEG_W,), f32)
        return 0

      lax.fori_loop(0, CH, fill_ones, 0)

    plsc.subcore_barrier()

    def gather(j, b):
      return pltpu.make_async_copy(y.at[src_v.at[j]], rows_v.at[b], gsem)

    def scat_start(j, b):
      pltpu.async_copy(rows_v.at[b], acc.at[dst_v.at[j]], ssem, add=True)
      if with_deg:
        @pl.when(c == (b % 2))
        def _():
          pltpu.sync_copy(ones_v, accd.at[dst_v.at[j]], add=True)

    def scat_wait(j, b):
      pltpu.make_async_copy(rows_v.at[b], acc.at[dst_v.at[j]], ssem).wait()

    for b in range(4):
      gather(b, b).start()

    def ring(jj, _):
      j0 = 4 * jj
      for b in range(4):
        gather(j0 + b, b).wait()
        scat_start(j0 + b, b)
      for b in range(4):
        scat_wait(j0 + b, b)
        gather(j0 + 4 + b, b).start()
      return 0

    NRING = (NCHUNK - 6) // 4
    lax.fori_loop(0, NRING, ring, 0)

    t0 = NRING * 4
    for b in range(4):
      gather(t0 + b, b).wait()
      scat_start(t0 + b, b)
    scat_wait(t0, 0)
    gather(t0 + 4, 0).start()
    scat_wait(t0 + 1, 1)
    gather(t0 + 5, 1).start()
    gather(t0 + 4, 0).wait()
    scat_start(t0 + 4, 0)
    gather(t0 + 5, 1).wait()
    scat_start(t0 + 5, 1)
    scat_wait(t0 + 2, 2)
    scat_wait(t0 + 3, 3)
    scat_wait(t0 + 4, 0)
    scat_wait(t0 + 5, 1)

    plsc.subcore_barrier()

    rs = s * RPT
    pltpu.sync_copy(acc.at[pl.ds(rs, RPT)], outp.at[c, pl.ds(rs, RPT)])

    @pl.when(s == 0)
    def _():
      pltpu.sync_copy(acc.at[pl.ds(NS * RPT, TAIL)],
                      outp.at[c, pl.ds(NS * RPT, TAIL)])

    if with_deg:
      pltpu.sync_copy(accd.at[pl.ds(rs, RPT)], outd.at[c, pl.ds(rs, RPT)])

      @pl.when(s == 0)
      def _():
        pltpu.sync_copy(accd.at[pl.ds(NS * RPT, TAIL)],
                        outd.at[c, pl.ds(NS * RPT, TAIL)])

  return pl.kernel(body, out_type=out_type, mesh=mesh, scratch_types=scratch,
                   compiler_params=pltpu.CompilerParams(
                       use_tc_tiling_on_sc=False),
                   name="sc_agg_deg" if with_deg else "sc_agg")


_sc_agg_deg = _make_sc_agg(True)
_sc_agg = _make_sc_agg(False)


BR = 1000


def _tc_first(x, ws, wn, b, s_out, y_out):
  xv = x[...]
  s_out[...] = jnp.dot(xv, ws[...], preferred_element_type=f32) + b[...]
  y_out[...] = jnp.dot(xv, wn[...], preferred_element_type=f32)


def _mean_from_parts(p, dr):
  deg = dr[0, :, 0:1] + dr[1, :, 0:1]
  inv = 1.0 / jnp.maximum(deg, 1.0)
  agg = jnp.concatenate([p[0], p[1]], axis=1)
  return agg * inv


def _tc_mid(s_in, p, dr, ws, wn, b, s_out, y_out):
  h = jnp.maximum(s_in[...] + _mean_from_parts(p, dr), 0.0)
  s_out[...] = jnp.dot(h, ws[...], preferred_element_type=f32) + b[...]
  y_out[...] = jnp.dot(h, wn[...], preferred_element_type=f32)


def _tc_last(s_in, p, dr, out):
  out[...] = s_in[...] + _mean_from_parts(p, dr)


_row_spec = pl.BlockSpec((BR, D), lambda i: (i, 0))
_w_spec = pl.BlockSpec((D, D), lambda i: (0, 0))
_b_spec = pl.BlockSpec((1, D), lambda i: (0, 0))
_p_spec = pl.BlockSpec((NC, BR, DH), lambda i: (0, i, 0))
_dr_spec = pl.BlockSpec((NC, BR, DEG_W), lambda i: (0, i, 0))
_grid = (N_NODES // BR,)
_nd_shape = jax.ShapeDtypeStruct((N_NODES, D), f32)

_tc_first_call = pl.pallas_call(
    _tc_first, grid=_grid,
    in_specs=[_row_spec, _w_spec, _w_spec, _b_spec],
    out_specs=[_row_spec, _row_spec],
    out_shape=[_nd_shape, _nd_shape])

_tc_mid_call = pl.pallas_call(
    _tc_mid, grid=_grid,
    in_specs=[_row_spec, _p_spec, _dr_spec, _w_spec, _w_spec, _b_spec],
    out_specs=[_row_spec, _row_spec],
    out_shape=[_nd_shape, _nd_shape])

_tc_last_call = pl.pallas_call(
    _tc_last, grid=_grid,
    in_specs=[_row_spec, _p_spec, _dr_spec],
    out_specs=_row_spec,
    out_shape=_nd_shape)


@jax.jit
def kernel(in_feat, edge_index, W_self1, W_neigh1, b1, W_self2, W_neigh2, b2,
           W_self3, W_neigh3, b3, W_self4, W_neigh4, b4):
  src = edge_index[0].reshape(ECH, CH)
  dst = edge_index[1].reshape(ECH, CH)
  s1, y1 = _tc_first_call(in_feat, W_self1, W_neigh1, b1.reshape(1, D))
  p, dr = _sc_agg_deg(y1.reshape(2 * N_NODES, DH), src, dst)
  s2, y2 = _tc_mid_call(s1, p, dr, W_self2, W_neigh2, b2.reshape(1, D))
  (p,) = _sc_agg(y2.reshape(2 * N_NODES, DH), src, dst)
  s3, y3 = _tc_mid_call(s2, p, dr, W_self3, W_neigh3, b3.reshape(1, D))
  (p,) = _sc_agg(y3.reshape(2 * N_NODES, DH), src, dst)
  s4, y4 = _tc_mid_call(s3, p, dr, W_self4, W_neigh4, b4.reshape(1, D))
  (p,) = _sc_agg(y4.reshape(2 * N_NODES, DH), src, dst)
  return _tc_last_call(s4, p, dr)


# speedup vs baseline: 1.3499x; 1.0652x over previous
import functools

import jax
import jax.numpy as jnp
from jax import lax
from jax.experimental import pallas as pl
from jax.experimental.pallas import tpu as pltpu
from jax.experimental.pallas import tpu_sc as plsc

N_NODES = 10000
N_EDGES = 320000
D = 128

NC = 2
NS = 16
DH = D // NC
EPW = N_EDGES // NS
CH = 80
NCHUNK = EPW // CH
NPAIR = NCHUNK // 2
ECH = N_EDGES // CH
RPT = 624
TAIL = N_NODES - NS * RPT
NBUF = 7
DEG_W = 16

f32 = jnp.float32


def _make_sc_agg(with_deg):
  out_type = [jax.ShapeDtypeStruct((NC, N_NODES, DH), f32)]
  scratch = [
      pltpu.VMEM((NCHUNK, CH), jnp.int32),
      pltpu.VMEM((NCHUNK, CH), jnp.int32),
      pltpu.VMEM((NBUF, CH, DH), f32),
      pltpu.VMEM_SHARED((N_NODES, DH), f32),
      pltpu.SemaphoreType.DMA,
      pltpu.SemaphoreType.DMA,
  ]
  if with_deg:
    out_type.append(jax.ShapeDtypeStruct((NC, N_NODES, DEG_W), f32))
    scratch += [
        pltpu.VMEM((CH, DEG_W), f32),
        pltpu.VMEM_SHARED((N_NODES, DEG_W), f32),
    ]

  mesh = plsc.VectorSubcoreMesh(core_axis_name="c", subcore_axis_name="s")

  def body(*refs):
    if with_deg:
      (y, srce, dste, z64, z16, outp, outd,
       src_v, dst_v, rows_v, acc, gsem, ssem, ones_v, accd) = refs
    else:
      (y, srce, dste, z64, outp,
       src_v, dst_v, rows_v, acc, gsem, ssem) = refs

    c = lax.axis_index("c")
    s = lax.axis_index("s")

    stage_s = pltpu.make_async_copy(srce.at[pl.ds(s * NCHUNK, NCHUNK)],
                                    src_v, gsem)
    stage_d = pltpu.make_async_copy(dste.at[pl.ds(s * NCHUNK, NCHUNK)],
                                    dst_v, gsem)
    stage_s.start()
    stage_d.start()

    pltpu.sync_copy(z64, acc.at[pl.ds(s * RPT, RPT)])

    @pl.when(s == 0)
    def _():
      pltpu.sync_copy(z64.at[pl.ds(0, TAIL)], acc.at[pl.ds(NS * RPT, TAIL)])

    if with_deg:
      pltpu.sync_copy(z16, accd.at[pl.ds(s * RPT, RPT)])

      @pl.when(s == 0)
      def _():
        pltpu.sync_copy(z16.at[pl.ds(0, TAIL)], accd.at[pl.ds(NS * RPT, TAIL)])

      def fill_ones(i, _):
        ones_v[i] = jnp.ones((DEG_W,), f32)
        return 0

      lax.fori_loop(0, CH, fill_ones, 0)

    stage_s.wait()
    stage_d.wait()
    npv = CH // 16

    def remap(t, _):
      i = t // npv
      k = t % npv
      v = src_v[i, pl.ds(k * 16, 16)]
      src_v[i, pl.ds(k * 16, 16)] = v * 2 + c
      return 0

    lax.fori_loop(0, NCHUNK * npv, remap, 0, unroll=npv)

    plsc.subcore_barrier()

    def gather(j, b):
      return pltpu.make_async_copy(y.at[src_v.at[j]], rows_v.at[b], gsem)

    def scat_start(j, b):
      pltpu.async_copy(rows_v.at[b], acc.at[dst_v.at[j]], ssem, add=True)
      if with_deg:
        @pl.when(c == (b % 2))
        def _():
          pltpu.sync_copy(ones_v, accd.at[dst_v.at[j]], add=True)

    def scat_wait(j, b):
      pltpu.make_async_copy(rows_v.at[b], acc.at[dst_v.at[j]], ssem).wait()

    for b in range(NBUF):
      gather(b, b).start()

    def ring(jj, _):
      j0 = NBUF * jj
      for b in range(NBUF):
        gather(j0 + b, b).wait()
        scat_start(j0 + b, b)
      for b in range(NBUF):
        scat_wait(j0 + b, b)
        gather(j0 + NBUF + b, b).start()
      return 0

    NRING = (NCHUNK - NBUF) // NBUF
    lax.fori_loop(0, NRING, ring, 0)

    t0 = NRING * NBUF
    NTAIL = NCHUNK - t0
    for b in range(NBUF):
      gather(t0 + b, b).wait()
      scat_start(t0 + b, b)
    for e in range(NTAIL - NBUF):
      scat_wait(t0 + e, e)
      gather(t0 + NBUF + e, e).start()
    for e in range(NTAIL - NBUF):
      gather(t0 + NBUF + e, e).wait()
      scat_start(t0 + NBUF + e, e)
    for b in range(NTAIL - NBUF, NBUF):
      scat_wait(t0 + b, b)
    for e in range(NTAIL - NBUF):
      scat_wait(t0 + NBUF + e, e)

    plsc.subcore_barrier()

    rs = s * RPT
    pltpu.sync_copy(acc.at[pl.ds(rs, RPT)], outp.at[c, pl.ds(rs, RPT)])

    @pl.when(s == 0)
    def _():
      pltpu.sync_copy(acc.at[pl.ds(NS * RPT, TAIL)],
                      outp.at[c, pl.ds(NS * RPT, TAIL)])

    if with_deg:
      pltpu.sync_copy(accd.at[pl.ds(rs, RPT)], outd.at[c, pl.ds(rs, RPT)])

      @pl.when(s == 0)
      def _():
        pltpu.sync_copy(accd.at[pl.ds(NS * RPT, TAIL)],
                        outd.at[c, pl.ds(NS * RPT, TAIL)])

  return pl.kernel(body, out_type=out_type, mesh=mesh, scratch_types=scratch,
                   compiler_params=pltpu.CompilerParams(
                       use_tc_tiling_on_sc=False),
                   name="sc_agg_deg" if with_deg else "sc_agg")


_sc_agg_deg = _make_sc_agg(True)
_sc_agg = _make_sc_agg(False)


BR = 1000


def _tc_first(x, ws, wn, b, s_out, y_out):
  xv = x[...]
  s_out[...] = jnp.dot(xv, ws[...], preferred_element_type=f32) + b[...]
  y_out[...] = jnp.dot(xv, wn[...], preferred_element_type=f32)


def _mean_from_parts(p, dr):
  deg = dr[0, :, 0:1] + dr[1, :, 0:1]
  inv = 1.0 / jnp.maximum(deg, 1.0)
  agg = jnp.concatenate([p[0], p[1]], axis=1)
  return agg * inv


def _tc_mid(s_in, p, dr, ws, wn, b, s_out, y_out):
  h = jnp.maximum(s_in[...] + _mean_from_parts(p, dr), 0.0)
  s_out[...] = jnp.dot(h, ws[...], preferred_element_type=f32) + b[...]
  y_out[...] = jnp.dot(h, wn[...], preferred_element_type=f32)


def _tc_last(s_in, p, dr, out):
  out[...] = s_in[...] + _mean_from_parts(p, dr)


_row_spec = pl.BlockSpec((BR, D), lambda i: (i, 0))
_w_spec = pl.BlockSpec((D, D), lambda i: (0, 0))
_b_spec = pl.BlockSpec((1, D), lambda i: (0, 0))
_p_spec = pl.BlockSpec((NC, BR, DH), lambda i: (0, i, 0))
_dr_spec = pl.BlockSpec((NC, BR, DEG_W), lambda i: (0, i, 0))
_grid = (N_NODES // BR,)
_nd_shape = jax.ShapeDtypeStruct((N_NODES, D), f32)

_tc_first_call = pl.pallas_call(
    _tc_first, grid=_grid,
    in_specs=[_row_spec, _w_spec, _w_spec, _b_spec],
    out_specs=[_row_spec, _row_spec],
    out_shape=[_nd_shape, _nd_shape])

_tc_mid_call = pl.pallas_call(
    _tc_mid, grid=_grid,
    in_specs=[_row_spec, _p_spec, _dr_spec, _w_spec, _w_spec, _b_spec],
    out_specs=[_row_spec, _row_spec],
    out_shape=[_nd_shape, _nd_shape])

_tc_last_call = pl.pallas_call(
    _tc_last, grid=_grid,
    in_specs=[_row_spec, _p_spec, _dr_spec],
    out_specs=_row_spec,
    out_shape=_nd_shape)


@jax.jit
def kernel(in_feat, edge_index, W_self1, W_neigh1, b1, W_self2, W_neigh2, b2,
           W_self3, W_neigh3, b3, W_self4, W_neigh4, b4):
  src = edge_index[0].reshape(ECH, CH)
  dst = edge_index[1].reshape(ECH, CH)
  s1, y1 = _tc_first_call(in_feat, W_self1, W_neigh1, b1.reshape(1, D))
  z64 = jnp.zeros((RPT, DH), f32)
  z16 = jnp.zeros((RPT, DEG_W), f32)
  p, dr = _sc_agg_deg(y1.reshape(2 * N_NODES, DH), src, dst, z64, z16)
  s2, y2 = _tc_mid_call(s1, p, dr, W_self2, W_neigh2, b2.reshape(1, D))
  (p,) = _sc_agg(y2.reshape(2 * N_NODES, DH), src, dst, z64)
  s3, y3 = _tc_mid_call(s2, p, dr, W_self3, W_neigh3, b3.reshape(1, D))
  (p,) = _sc_agg(y3.reshape(2 * N_NODES, DH), src, dst, z64)
  s4, y4 = _tc_mid_call(s3, p, dr, W_self4, W_neigh4, b4.reshape(1, D))
  (p,) = _sc_agg(y4.reshape(2 * N_NODES, DH), src, dst, z64)
  return _tc_last_call(s4, p, dr)
